# baseline (speedup 1.0000x reference)
import functools

import jax
import jax.numpy as jnp
from jax import lax
from jax.experimental import pallas as pl
from jax.experimental.pallas import tpu as pltpu
from jax.experimental.pallas import tpu_sc as plsc

_K = 8
_L = 16
_CH = 16
_NC = 2
_NS = 16
_NW = _NC * _NS


def _sort_desc(v):
    s, _ = plsc.sort_key_val(v, v, descending=True)
    return s


def _merge_desc(a, b):
    t = jnp.maximum(a, lax.rev(b, (0,)))
    return _sort_desc(t)


def _insert(ms, v):
    out = []
    t = v
    for mk in ms:
        out.append(jnp.maximum(mk, t))
        t = jnp.minimum(mk, t)
    return tuple(out)


def kernel(scores):
    rows, n = scores.shape
    rows_per_w = rows // _NW
    nvec = n // _L
    mesh = plsc.VectorSubcoreMesh(core_axis_name="c", subcore_axis_name="s")

    @functools.partial(
        pl.kernel,
        out_type=jax.ShapeDtypeStruct((rows, _L), jnp.float32),
        mesh=mesh,
        scratch_types=[
            pltpu.VMEM((n,), jnp.float32),
            pltpu.VMEM((n,), jnp.float32),
            pltpu.VMEM((n // _L,), jnp.float32),
            pltpu.VMEM((rows_per_w, _L), jnp.float32),
            pltpu.SemaphoreType.DMA,
            pltpu.SemaphoreType.DMA,
        ],
        compiler_params=pltpu.CompilerParams(needs_layout_passes=False),
    )
    def _topk(scores_hbm, out_hbm, rowbuf0, rowbuf1, cmaxbuf, outbuf, sem0, sem1):
        cid = lax.axis_index("c")
        sid = lax.axis_index("s")
        base = (sid * _NC + cid) * rows_per_w
        bufs = (rowbuf0, rowbuf1)
        sems = (sem0, sem1)

        pltpu.make_async_copy(scores_hbm.at[base], rowbuf0, sem0).start()
        for r in range(rows_per_w):
            cur = r % 2
            rb = bufs[cur]
            with jax.named_scope("dma_wait"):
                pltpu.make_async_copy(scores_hbm.at[base + r], rb, sems[cur]).wait()
            if r + 1 < rows_per_w:
                nxt = (r + 1) % 2
                pltpu.make_async_copy(
                    scores_hbm.at[base + r + 1], bufs[nxt], sems[nxt]
                ).start()

            neg = jnp.full((_L,), -jnp.inf, jnp.float32)
            init = tuple(neg for _ in range(_K))

            def chunk_max(c, m):
                bv = c * (_CH * _L)
                vs = [rb[pl.ds(bv + j * _L, _L)] for j in range(_CH)]
                while len(vs) > 1:
                    vs = [
                        jnp.maximum(vs[2 * j], vs[2 * j + 1])
                        for j in range(len(vs) // 2)
                    ]
                cmaxbuf[pl.ds(c * _L, _L)] = vs[0]
                return jnp.maximum(m, vs[0])

            nch = nvec // _CH
            with jax.named_scope("passA"):
                m = lax.fori_loop(0, nch, chunk_max, neg)

            sm = _sort_desc(m)
            lane = lax.iota(jnp.int32, _L)
            tau = lax.reduce_max(
                jnp.where(lane >= _K - 1, sm, -jnp.inf), (0,)
            )

            def scan_chunk(c, ms):
                cm = cmaxbuf[pl.ds(c * _L, _L)]
                hit = jnp.any(cm >= tau)

                def do_insert():
                    out = ms
                    bv = c * (_CH * _L)
                    for j in range(_CH):
                        out = _insert(out, rb[pl.ds(bv + j * _L, _L)])
                    return out

                return lax.cond(hit, do_insert, lambda: ms)

            with jax.named_scope("passB"):
                ms = lax.fori_loop(0, nch, scan_chunk, init)

            with jax.named_scope("select"):
                s = [_sort_desc(m) for m in ms]
                s = [_merge_desc(s[2 * j], s[2 * j + 1]) for j in range(4)]
                s = [_merge_desc(s[0], s[1]), _merge_desc(s[2], s[3])]
                outbuf[r, :] = _merge_desc(s[0], s[1])

        pltpu.sync_copy(outbuf, out_hbm.at[pl.ds(base, rows_per_w)])

    out = _topk(scores)
    return out[:, :_K]


# speedup vs baseline: 1.1714x; 1.1714x over previous
import functools

import jax
import jax.numpy as jnp
from jax import lax
from jax.experimental import pallas as pl
from jax.experimental.pallas import tpu as pltpu
from jax.experimental.pallas import tpu_sc as plsc

_K = 8
_L = 16
_CH = 16
_NC = 2
_NS = 16
_NW = _NC * _NS


def _sort_desc(v):
    s, _ = plsc.sort_key_val(v, v, descending=True)
    return s


def _merge_desc(a, b):
    t = jnp.maximum(a, lax.rev(b, (0,)))
    return _sort_desc(t)


def _insert(ms, v):
    out = []
    t = v
    for mk in ms:
        out.append(jnp.maximum(mk, t))
        t = jnp.minimum(mk, t)
    return tuple(out)


def kernel(scores):
    rows, n = scores.shape
    rows_per_w = rows // _NW
    nvec = n // _L
    mesh = plsc.VectorSubcoreMesh(core_axis_name="c", subcore_axis_name="s")

    @functools.partial(
        pl.kernel,
        out_type=jax.ShapeDtypeStruct((rows, _L), jnp.float32),
        mesh=mesh,
        scratch_types=[
            pltpu.VMEM((n,), jnp.float32),
            pltpu.VMEM((n,), jnp.float32),
            pltpu.VMEM((n // _L,), jnp.float32),
            pltpu.VMEM((n // (_L * _CH) + _L,), jnp.int32),
            pltpu.VMEM((rows_per_w, _L), jnp.float32),
            pltpu.SemaphoreType.DMA,
            pltpu.SemaphoreType.DMA,
        ],
        compiler_params=pltpu.CompilerParams(needs_layout_passes=False),
    )
    def _topk(
        scores_hbm, out_hbm, rowbuf0, rowbuf1, cmaxbuf, hitbuf, outbuf, sem0, sem1
    ):
        cid = lax.axis_index("c")
        sid = lax.axis_index("s")
        base = (sid * _NC + cid) * rows_per_w
        bufs = (rowbuf0, rowbuf1)
        sems = (sem0, sem1)

        pltpu.make_async_copy(scores_hbm.at[base], rowbuf0, sem0).start()
        for r in range(rows_per_w):
            cur = r % 2
            rb = bufs[cur]
            with jax.named_scope("dma_wait"):
                pltpu.make_async_copy(scores_hbm.at[base + r], rb, sems[cur]).wait()
            if r + 1 < rows_per_w:
                nxt = (r + 1) % 2
                pltpu.make_async_copy(
                    scores_hbm.at[base + r + 1], bufs[nxt], sems[nxt]
                ).start()

            neg = jnp.full((_L,), -jnp.inf, jnp.float32)
            init = tuple(neg for _ in range(_K))

            def chunk_max(c, m):
                bv = c * (_CH * _L)
                vs = [rb[pl.ds(bv + j * _L, _L)] for j in range(_CH)]
                while len(vs) > 1:
                    vs = [
                        jnp.maximum(vs[2 * j], vs[2 * j + 1])
                        for j in range(len(vs) // 2)
                    ]
                cmaxbuf[pl.ds(c * _L, _L)] = vs[0]
                return jnp.maximum(m, vs[0])

            nch = nvec // _CH
            with jax.named_scope("passA"):
                m = lax.fori_loop(0, nch, chunk_max, neg)

            sm = _sort_desc(m)
            lane = lax.iota(jnp.int32, _L)
            tau = lax.reduce_max(
                jnp.where(lane >= _K - 1, sm, -jnp.inf), (0,)
            )

            lane0 = lane == 0

            def find_hits(c, ptr):
                cm = cmaxbuf[pl.ds(c * _L, _L)]
                pop = plsc.all_reduce_population_count(cm >= tau)
                hit = pop > 0
                plsc.store_scatter(
                    hitbuf, [ptr], jnp.full((_L,), 0, jnp.int32) + c,
                    mask=hit & lane0,
                )
                return ptr + jnp.where(hit, 1, 0)

            with jax.named_scope("passB1"):
                zero = jnp.zeros((_L,), jnp.int32)
                ptr = lax.fori_loop(0, nch, find_hits, zero)
                nhit = lax.reduce_max(ptr, (0,))

            def insert_hit(i, ms):
                c = hitbuf[pl.ds(i, _L)][0]
                bv = c * (_CH * _L)
                out = ms
                for j in range(_CH):
                    out = _insert(out, rb[pl.ds(bv + j * _L, _L)])
                return out

            with jax.named_scope("passB2"):
                ms = lax.fori_loop(0, nhit, insert_hit, init)

            with jax.named_scope("select"):
                s = [_sort_desc(m) for m in ms]
                s = [_merge_desc(s[2 * j], s[2 * j + 1]) for j in range(4)]
                s = [_merge_desc(s[0], s[1]), _merge_desc(s[2], s[3])]
                outbuf[r, :] = _merge_desc(s[0], s[1])

        pltpu.sync_copy(outbuf, out_hbm.at[pl.ds(base, rows_per_w)])

    out = _topk(scores)
    return out[:, :_K]
